# Initial kernel scaffold; baseline (speedup 1.0000x reference)
import jax
import jax.numpy as jnp
from jax.experimental import pallas as pl


def kernel(x, W):
    raise NotImplementedError("write your pallas kernel here")


# speedup vs baseline: 1.7811x; 1.7811x over previous
import functools

import jax
import jax.numpy as jnp
from jax.experimental import pallas as pl
from jax.experimental.pallas import tpu as pltpu

INPUT_DIM = 2048
NUM_ROUTED = 16
TOTAL = NUM_ROUTED + 1
OUT_PAD = 32
B, S = 4, 4096
N_TOKENS = B * S
BLOCK_ROWS = 1024
N_BLOCKS = N_TOKENS // BLOCK_ROWS


def _gate_kernel(x_ref, w_ref, g_ref, dm_ref, loss_ref, stats_ref):
    i = pl.program_id(0)
    logits = jax.lax.dot_general(
        x_ref[...], w_ref[...],
        dimension_numbers=(((1,), (1,)), ((), ())),
        preferred_element_type=jnp.float32,
    ) + g_ref[...]
    m = jnp.max(logits, axis=-1, keepdims=True)
    e = jnp.exp(logits - m)
    s = e / jnp.sum(e, axis=-1, keepdims=True)

    iota = jax.lax.broadcasted_iota(jnp.int32, s.shape, 1)
    v1 = jnp.max(s, axis=-1, keepdims=True)
    i1 = jnp.min(jnp.where(s == v1, iota, NUM_ROUTED), axis=-1, keepdims=True)
    s2 = jnp.where(iota == i1, -1.0, s)
    v2 = jnp.max(s2, axis=-1, keepdims=True)
    i2 = jnp.min(jnp.where(s2 == v2, iota, NUM_ROUTED), axis=-1, keepdims=True)

    lane = jax.lax.broadcasted_iota(jnp.int32, (s.shape[0], OUT_PAD), 1)
    eid = lane - 1
    dm = jnp.where(eid == i1, v1, jnp.where(eid == i2, v2, 0.0))
    dm_ref[...] = jnp.where(lane == 0, 1.0, dm)

    ssum = jnp.sum(s, axis=0)
    sqsum = jnp.sum(s * s, axis=0)
    block = jnp.concatenate(
        [ssum[None, :], sqsum[None, :], jnp.zeros((6, NUM_ROUTED), jnp.float32)], axis=0)

    @pl.when(i == 0)
    def _():
        stats_ref[...] = block

    @pl.when(i > 0)
    def _():
        stats_ref[...] = stats_ref[...] + block

    @pl.when(i == N_BLOCKS - 1)
    def _():
        tot = stats_ref[...]
        me = tot[0, :] / N_TOKENS
        ce = tot[1, :] / N_TOKENS
        loss_ref[...] = jnp.sum(me * ce).reshape(1, 1) * (NUM_ROUTED ** 2)


@functools.partial(jax.jit, static_argnames=("interpret",))
def kernel(x, W, interpret=False):
    noise = jax.random.uniform(jax.random.key(1234), (B, S, NUM_ROUTED),
                               dtype=jnp.float32)
    gumbel = -jnp.log(-jnp.log(noise + 1e-9) + 1e-9)
    g2 = gumbel.reshape(N_TOKENS, NUM_ROUTED)
    x2 = x.reshape(N_TOKENS, INPUT_DIM)

    dm, loss = pl.pallas_call(
        _gate_kernel,
        grid=(N_BLOCKS,),
        in_specs=[
            pl.BlockSpec((BLOCK_ROWS, INPUT_DIM), lambda i: (i, 0)),
            pl.BlockSpec((NUM_ROUTED, INPUT_DIM), lambda i: (0, 0)),
            pl.BlockSpec((BLOCK_ROWS, NUM_ROUTED), lambda i: (i, 0)),
        ],
        out_specs=[
            pl.BlockSpec((BLOCK_ROWS, OUT_PAD), lambda i: (i, 0)),
            pl.BlockSpec((1, 1), lambda i: (0, 0)),
        ],
        out_shape=[
            jax.ShapeDtypeStruct((N_TOKENS, OUT_PAD), jnp.float32),
            jax.ShapeDtypeStruct((1, 1), jnp.float32),
        ],
        scratch_shapes=[pltpu.VMEM((8, NUM_ROUTED), jnp.float32)],
        interpret=interpret,
    )(x2, W, g2)

    dispatch = dm[:, :TOTAL].reshape(B, S, TOTAL)
    return dispatch, loss[0, 0]
